# baseline (speedup 1.0000x reference)
import functools
import jax
import jax.numpy as jnp
from jax import lax
from jax.experimental import pallas as pl
from jax.experimental.pallas import tpu as pltpu
from jax.experimental.pallas import tpu_sc as plsc

_BPB = 4


def _dist_body(zb, cb, csq_ref, idx_slot_ref):
    zf = zb.T
    zsq = jnp.sum(zb * zb, axis=0)[:, None]
    HW_ = zf.shape[0]
    K_ = cb.shape[0]
    CH = 256
    pval = None
    pj = None
    for j in range(K_ // CH):
        cbj = cb[j * CH:(j + 1) * CH, :]
        s = jax.lax.dot_general(
            zf, cbj, (((1,), (1,)), ((), ())),
            preferred_element_type=jnp.float32)
        dj = (zsq - 2.0 * s) + csq_ref[:, j * CH:(j + 1) * CH]
        if j == 0:
            pval = dj
            pj = jnp.zeros((HW_, CH), jnp.int32)
        else:
            lt = dj < pval
            pval = jnp.where(lt, dj, pval)
            pj = jnp.where(lt, jnp.int32(j), pj)
    lane = jax.lax.broadcasted_iota(jnp.int32, (HW_, CH), 1)
    pkey = pj * CH + lane
    tval = pval.T
    tkey = pkey.T
    m = jnp.min(tval, axis=0)[None, :]
    big = jnp.int32(K_)
    idx = jnp.min(jnp.where(tval == m, tkey, big), axis=0)
    idx_slot_ref[...] = idx


def _dist_kernel(z_ref, cb_ref, idx_ref, csq_ref):
    cb = cb_ref[...]

    @pl.when(pl.program_id(0) == 0)
    def _init():
        csq_ref[...] = jnp.sum(cb * cb, axis=1, keepdims=True).T

    for t in range(_BPB):
        _dist_body(z_ref[t], cb, csq_ref, idx_ref.at[t, 0])


def _make_sc_gather(K, C, N, NC, NS):
    NW = NC * NS
    n_per_w = N // NW
    CHG = 128
    n_chunks = n_per_w // CHG
    mesh = plsc.VectorSubcoreMesh(core_axis_name="c", subcore_axis_name="s")

    @functools.partial(
        pl.kernel, mesh=mesh,
        out_type=jax.ShapeDtypeStruct((N, C), jnp.float32),
        scratch_types=[
            pltpu.VMEM((CHG,), jnp.int32),
            pltpu.VMEM((CHG, C), jnp.float32),
            pltpu.SemaphoreType.DMA,
        ],
    )
    def gather_k(cb_hbm, idx_hbm, out_hbm, idx_v, rows_v, sem):
        wid = lax.axis_index("s") * NC + lax.axis_index("c")
        base = wid * n_per_w
        for chunk in range(n_chunks):
            off = base + chunk * CHG
            pltpu.sync_copy(idx_hbm.at[pl.ds(off, CHG)], idx_v)
            pltpu.async_copy(cb_hbm.at[idx_v], rows_v, sem).wait()
            pltpu.sync_copy(rows_v, out_hbm.at[pl.ds(off, CHG)])

    return gather_k


def kernel(z, codebook):
    B, C, H, W = z.shape
    HW = H * W
    K = codebook.shape[0]
    zr = z.reshape(B, C, HW)
    info = plsc.get_sparse_core_info()

    ng = _BPB * HW
    gather_k = _make_sc_gather(K, C, ng, info.num_cores, info.num_subcores)

    idx_parts = []
    zq_parts = []
    for g in range(B // _BPB):
        zg = jax.lax.slice_in_dim(zr, g * _BPB, (g + 1) * _BPB, axis=0)
        idx3 = pl.pallas_call(
            _dist_kernel,
            grid=(1,),
            in_specs=[
                pl.BlockSpec((_BPB, C, HW), lambda b: (0, 0, 0)),
                pl.BlockSpec((K, C), lambda b: (0, 0)),
            ],
            out_specs=pl.BlockSpec((_BPB, 1, HW), lambda b: (0, 0, 0)),
            out_shape=jax.ShapeDtypeStruct((_BPB, 1, HW), jnp.int32),
            scratch_shapes=[pltpu.VMEM((1, K), jnp.float32)],
        )(zg, codebook)
        zqf = gather_k(codebook, idx3.reshape(ng))
        idx_parts.append(idx3.reshape(_BPB, HW))
        zq_parts.append(zqf.reshape(_BPB, HW, C).transpose(0, 2, 1))

    zq = jnp.concatenate(zq_parts, axis=0).reshape(B, C, H, W)
    idx = jnp.concatenate(idx_parts, axis=0)
    return zq, idx


# speedup vs baseline: 1.4928x; 1.4928x over previous
import functools
import jax
import jax.numpy as jnp
from jax import lax
from jax.experimental import pallas as pl
from jax.experimental.pallas import tpu as pltpu
from jax.experimental.pallas import tpu_sc as plsc

_BPB = 4


def _dist_body(zb, cb, csq_ref, idx_slot_ref):
    zf = zb.T
    zsq = jnp.sum(zb * zb, axis=0)[:, None]
    HW_ = zf.shape[0]
    K_ = cb.shape[0]
    CH = 256
    pval = None
    pj = None
    for j in range(K_ // CH):
        cbj = cb[j * CH:(j + 1) * CH, :]
        s = jax.lax.dot_general(
            zf, cbj, (((1,), (1,)), ((), ())),
            preferred_element_type=jnp.float32)
        dj = (zsq - 2.0 * s) + csq_ref[:, j * CH:(j + 1) * CH]
        if j == 0:
            pval = dj
            pj = jnp.zeros((HW_, CH), jnp.int32)
        else:
            lt = dj < pval
            pval = jnp.where(lt, dj, pval)
            pj = jnp.where(lt, jnp.int32(j), pj)
    lane = jax.lax.broadcasted_iota(jnp.int32, (HW_, CH), 1)
    pkey = pj * CH + lane
    tval = pval.T
    tkey = pkey.T
    m = jnp.min(tval, axis=0)[None, :]
    big = jnp.int32(K_)
    idx = jnp.min(jnp.where(tval == m, tkey, big), axis=0)
    idx_slot_ref[...] = idx


def _dist_kernel(z_ref, cb_ref, idx_ref, csq_ref):
    cb = cb_ref[...]

    @pl.when(pl.program_id(0) == 0)
    def _init():
        csq_ref[...] = jnp.sum(cb * cb, axis=1, keepdims=True).T

    for t in range(_BPB):
        _dist_body(z_ref[t], cb, csq_ref, idx_ref.at[t, 0])


def _make_sc_gather(K, C, N, NC, NS):
    NW = NC * NS
    n_per_w = N // NW
    CHG = 128
    n_chunks = n_per_w // CHG
    mesh = plsc.VectorSubcoreMesh(core_axis_name="c", subcore_axis_name="s")

    @functools.partial(
        pl.kernel, mesh=mesh,
        out_type=jax.ShapeDtypeStruct((N, C), jnp.float32),
        scratch_types=(
            [pltpu.VMEM((CHG,), jnp.int32) for _ in range(n_chunks)]
            + [pltpu.VMEM((CHG, C), jnp.float32) for _ in range(2)]
            + [pltpu.SemaphoreType.DMA for _ in range(2)]
        ),
    )
    def gather_k(cb_hbm, idx_hbm, out_hbm, *scr):
        idx_v = scr[:n_chunks]
        bufs = scr[n_chunks:n_chunks + 2]
        gsem = scr[n_chunks + 2:n_chunks + 4]
        wid = lax.axis_index("s") * NC + lax.axis_index("c")
        base = wid * n_per_w
        for c in range(n_chunks):
            pltpu.sync_copy(idx_hbm.at[pl.ds(base + c * CHG, CHG)], idx_v[c])
        gathers = [None] * n_chunks
        stores = [None] * n_chunks
        gathers[0] = pltpu.make_async_copy(
            cb_hbm.at[idx_v[0]], bufs[0], gsem[0])
        gathers[0].start()
        if n_chunks > 1:
            gathers[1] = pltpu.make_async_copy(
                cb_hbm.at[idx_v[1]], bufs[1], gsem[1])
            gathers[1].start()
        for c in range(n_chunks):
            gathers[c].wait()
            stores[c] = pltpu.make_async_copy(
                bufs[c % 2], out_hbm.at[pl.ds(base + c * CHG, CHG)],
                gsem[c % 2])
            stores[c].start()
            if c + 2 < n_chunks:
                stores[c].wait()
                gathers[c + 2] = pltpu.make_async_copy(
                    cb_hbm.at[idx_v[c + 2]], bufs[c % 2], gsem[c % 2])
                gathers[c + 2].start()
        for c in range(max(0, n_chunks - 2), n_chunks):
            stores[c].wait()

    return gather_k


def kernel(z, codebook):
    B, C, H, W = z.shape
    HW = H * W
    K = codebook.shape[0]
    N = B * HW
    zr = z.reshape(B, C, HW)
    nb = B // _BPB

    idx3 = pl.pallas_call(
        _dist_kernel,
        grid=(nb,),
        in_specs=[
            pl.BlockSpec((_BPB, C, HW), lambda b: (b, 0, 0)),
            pl.BlockSpec((K, C), lambda b: (0, 0)),
        ],
        out_specs=pl.BlockSpec((_BPB, 1, HW), lambda b: (b, 0, 0)),
        out_shape=jax.ShapeDtypeStruct((B, 1, HW), jnp.int32),
        scratch_shapes=[pltpu.VMEM((1, K), jnp.float32)],
    )(zr, codebook)

    info = plsc.get_sparse_core_info()
    gather_k = _make_sc_gather(K, C, N, info.num_cores, info.num_subcores)
    zq_flat = gather_k(codebook, idx3.reshape(N))

    zq = zq_flat.reshape(B, HW, C).transpose(0, 2, 1).reshape(B, C, H, W)
    return zq, idx3.reshape(B, HW)
